# baseline (speedup 1.0000x reference)
import jax
import jax.numpy as jnp
from jax.experimental import pallas as pl

H = 1024
F = 8192
NT = 2
KTOP = 983
BM_ENC = 256
BM_DEC = 256
BN_DEC = 1024
INT_MIN32 = -2147483648


def _enc_body(x_ref, g_ref, bt_ref, w_ref, be_ref, z_ref):
    x = x_ref[...]
    mean = jnp.mean(x, axis=1, keepdims=True)
    xc = x - mean
    var = jnp.mean(xc * xc, axis=1, keepdims=True)
    xn = xc * jax.lax.rsqrt(var + 1e-5)
    xn = xn * g_ref[...] + bt_ref[...]
    xh = xn.astype(jnp.bfloat16)
    zp = jnp.dot(xh, w_ref[...], preferred_element_type=jnp.float32)
    zp = zp + be_ref[...]
    def _keys():
        zb = jax.lax.bitcast_convert_type(zp, jnp.int32)
        return zb ^ (jax.lax.shift_right_arithmetic(zb, 31)
                     & jnp.int32(0x7FFFFFFF))

    top16 = jax.lax.shift_right_arithmetic(_keys(), 16).astype(jnp.int16)
    cnt0 = jnp.sum((top16 >= 0).astype(jnp.int16), axis=1, keepdims=True)
    p_lo0 = jnp.where(cnt0 >= KTOP, jnp.zeros_like(cnt0),
                      jnp.full_like(cnt0, jnp.int16(-32768)))
    step0 = jnp.full_like(cnt0, jnp.int16(16384))

    def b1(i, carry):
        lo, step = carry
        cand = lo + step
        cnt = jnp.sum((top16 >= cand).astype(jnp.int16), axis=1,
                      keepdims=True)
        return jnp.where(cnt >= KTOP, cand, lo), step >> 1

    p16, _ = jax.lax.fori_loop(0, 15, b1, (p_lo0, step0))
    cnt_gt = jnp.sum((top16 > p16).astype(jnp.int16), axis=1, keepdims=True)
    rk = jnp.int16(KTOP) - cnt_gt
    lob = (_keys() & jnp.int32(0xFFFF)) - jnp.int32(32768)
    lo16 = jnp.where(top16 == p16, lob.astype(jnp.int16), jnp.int16(-32768))
    cnt0b = jnp.sum((lo16 >= 0).astype(jnp.int16), axis=1, keepdims=True)
    c_lo0 = jnp.where(cnt0b >= rk, jnp.zeros_like(rk),
                      jnp.full_like(rk, jnp.int16(-32768)))

    def b2(i, carry):
        lo, step = carry
        cand = lo + step
        cnt = jnp.sum((lo16 >= cand).astype(jnp.int16), axis=1,
                      keepdims=True)
        return jnp.where(cnt >= rk, cand, lo), step >> 1

    c16, _ = jax.lax.fori_loop(0, 15, b2, (c_lo0, step0))
    lowbits = c16.astype(jnp.int32) + jnp.int32(32768)
    thr = (p16.astype(jnp.int32) << 16) | lowbits
    tv = thr ^ (jax.lax.shift_right_arithmetic(thr, 31)
                & jnp.int32(0x7FFFFFFF))
    thrv = jax.lax.bitcast_convert_type(tv, jnp.float32)
    z_ref[...] = jnp.where(zp >= thrv, jnp.maximum(zp, 0.0), 0.0)


def _dec_body(z_ref, w_ref, bd_ref, y_ref):
    zb16 = z_ref[...].astype(jnp.bfloat16)
    y = jnp.dot(zb16, w_ref[...], preferred_element_type=jnp.float32)
    y_ref[...] = y + bd_ref[...]


def kernel(x, gamma, beta, W_enc, b_enc, W_dec, b_dec):
    B, T, _ = x.shape
    N = B * T
    x2 = x.reshape(N, H)
    wh = W_enc.astype(jnp.bfloat16)
    g2 = gamma.reshape(1, H)
    bt2 = beta.reshape(1, H)
    be2 = b_enc.reshape(1, F)
    bd2 = b_dec.reshape(1, NT * H)
    wd16 = W_dec.astype(jnp.bfloat16)

    z = pl.pallas_call(
        _enc_body,
        grid=(N // BM_ENC,),
        in_specs=[
            pl.BlockSpec((BM_ENC, H), lambda m: (m, 0)),
            pl.BlockSpec((1, H), lambda m: (0, 0)),
            pl.BlockSpec((1, H), lambda m: (0, 0)),
            pl.BlockSpec((H, F), lambda m: (0, 0)),
            pl.BlockSpec((1, F), lambda m: (0, 0)),
        ],
        out_specs=pl.BlockSpec((BM_ENC, F), lambda m: (m, 0)),
        out_shape=jax.ShapeDtypeStruct((N, F), jnp.float32),
    )(x2, g2, bt2, wh, be2)

    DN = NT * H
    y = pl.pallas_call(
        _dec_body,
        grid=(DN // BN_DEC, N // BM_DEC),
        in_specs=[
            pl.BlockSpec((BM_DEC, F), lambda n, m: (m, 0)),
            pl.BlockSpec((F, BN_DEC), lambda n, m: (0, n)),
            pl.BlockSpec((1, BN_DEC), lambda n, m: (0, n)),
        ],
        out_specs=pl.BlockSpec((BM_DEC, BN_DEC), lambda n, m: (m, n)),
        out_shape=jax.ShapeDtypeStruct((N, DN), jnp.float32),
    )(z, wd16, bd2)

    return (y.reshape(B, T, NT, H), z.reshape(B, T, F))


# speedup vs baseline: 1.3546x; 1.3546x over previous
import jax
import jax.numpy as jnp
from jax.experimental import pallas as pl

H = 1024
F = 8192
NT = 2
KTOP = 983
BM_ENC = 256
BM_DEC = 256
BN_DEC = 1024
INT_MIN32 = -2147483648


def _enc_body(x_ref, g_ref, bt_ref, w_ref, be_ref, z_ref):
    x = x_ref[...]
    mean = jnp.mean(x, axis=1, keepdims=True)
    xc = x - mean
    var = jnp.mean(xc * xc, axis=1, keepdims=True)
    xn = xc * jax.lax.rsqrt(var + 1e-5)
    xn = xn * g_ref[...] + bt_ref[...]
    xh = xn.astype(jnp.bfloat16)
    zp = jnp.dot(xh, w_ref[...], preferred_element_type=jnp.float32)
    zp = zp + be_ref[...]
    zb = jax.lax.bitcast_convert_type(zp, jnp.int32)
    keys = jnp.where(zb < 0, zb ^ jnp.int32(0x7FFFFFFF), zb)
    cnt0 = jnp.sum((keys >= 0).astype(jnp.int32), axis=1, keepdims=True)
    lo0 = jnp.where(cnt0 >= KTOP, jnp.zeros_like(cnt0),
                    jnp.full_like(cnt0, jnp.int32(INT_MIN32)))

    def body(i, lo):
        cand = lo + jnp.left_shift(jnp.int32(1), 30 - i)
        cnt = jnp.sum((keys >= cand).astype(jnp.int32), axis=1, keepdims=True)
        return jnp.where(cnt >= KTOP, cand, lo)

    thr = jax.lax.fori_loop(0, 31, body, lo0)
    z_ref[...] = jnp.where(keys >= thr, jnp.maximum(zp, 0.0), 0.0)


def _dec_body(z_ref, w_ref, bd_ref, y_ref):
    zb16 = z_ref[...].astype(jnp.bfloat16)
    y = jnp.dot(zb16, w_ref[...], preferred_element_type=jnp.float32)
    y_ref[...] = y + bd_ref[...]


def kernel(x, gamma, beta, W_enc, b_enc, W_dec, b_dec):
    B, T, _ = x.shape
    N = B * T
    x2 = x.reshape(N, H)
    wh = W_enc.astype(jnp.bfloat16)
    g2 = gamma.reshape(1, H)
    bt2 = beta.reshape(1, H)
    be2 = b_enc.reshape(1, F)
    bd2 = b_dec.reshape(1, NT * H)
    wd16 = W_dec.astype(jnp.bfloat16)

    z = pl.pallas_call(
        _enc_body,
        grid=(N // BM_ENC,),
        in_specs=[
            pl.BlockSpec((BM_ENC, H), lambda m: (m, 0)),
            pl.BlockSpec((1, H), lambda m: (0, 0)),
            pl.BlockSpec((1, H), lambda m: (0, 0)),
            pl.BlockSpec((H, F), lambda m: (0, 0)),
            pl.BlockSpec((1, F), lambda m: (0, 0)),
        ],
        out_specs=pl.BlockSpec((BM_ENC, F), lambda m: (m, 0)),
        out_shape=jax.ShapeDtypeStruct((N, F), jnp.float32),
    )(x2, g2, bt2, wh, be2)

    DN = NT * H
    y = pl.pallas_call(
        _dec_body,
        grid=(N // BM_DEC,),
        in_specs=[
            pl.BlockSpec((BM_DEC, F), lambda m: (m, 0)),
            pl.BlockSpec((F, DN), lambda m: (0, 0)),
            pl.BlockSpec((1, DN), lambda m: (0, 0)),
        ],
        out_specs=pl.BlockSpec((BM_DEC, DN), lambda m: (m, 0)),
        out_shape=jax.ShapeDtypeStruct((N, DN), jnp.float32),
    )(z, wd16, bd2)

    return (y.reshape(B, T, NT, H), z.reshape(B, T, F))
